# Initial kernel scaffold; baseline (speedup 1.0000x reference)
import jax
import jax.numpy as jnp
from jax.experimental import pallas as pl


def kernel(x, W_A, keys_A, values_A, W_B, keys_B, values_B):
    raise NotImplementedError("write your pallas kernel here")


# speedup vs baseline: 46.3286x; 46.3286x over previous
import jax
import jax.numpy as jnp
from jax.experimental import pallas as pl

B = 2
N = 2048
D = 1024
R = 8
CB_IN = 16
C = 64
K = 64
OUT_C = 128
T = B * N
TT = 256
G = T // TT

_F32 = jnp.float32
_PREC = jax.lax.Precision.DEFAULT


def _dot(a, b, contract, batch):
    return jax.lax.dot_general(
        a, b, dimension_numbers=(contract, batch),
        preferred_element_type=_F32, precision=_PREC)


def _argmin_onehot(xT_t, Wf, keys):
    projT = _dot(Wf, xT_t, ((1,), (0,)), ((), ()))
    proj3 = projT.reshape(C, CB_IN, TT)
    pk = _dot(keys, proj3, ((2,), (1,)), ((0,), (0,)))
    knorm = jnp.sum(keys * keys, axis=2)
    d2 = knorm[:, :, None] - 2.0 * pk
    idx = jnp.argmin(d2, axis=1)
    kio = jax.lax.broadcasted_iota(jnp.int32, (C, K, TT), 1)
    return (kio == idx[:, None, :]).astype(_F32)


def _tile_kernel(xT_ref, WAf_ref, keysA_ref, vAT_ref,
                 WBf_ref, keysB_ref, vBg_ref, out_ref):
    xT_t = xT_ref[...]

    onehotA = _argmin_onehot(xT_t, WAf_ref[...], keysA_ref[...])
    gAT = _dot(vAT_ref[...], onehotA, ((2,), (1,)), ((0,), (0,)))
    xseg = xT_t.reshape(R, OUT_C, TT)
    t_vals = jnp.sum(gAT.reshape(R, R, OUT_C, TT) * xseg[None], axis=(1, 2))

    onehotB = _argmin_onehot(xT_t, WBf_ref[...], keysB_ref[...])

    coef = (onehotB.reshape(R, R, K, TT) * t_vals[:, None, None, :])
    coef = coef.transpose(1, 0, 2, 3).reshape(R, R * K, TT)
    outT = _dot(vBg_ref[...], coef, ((2,), (1,)), ((0,), (0,)))
    out_ref[...] = outT.reshape(D, TT)


def kernel(x, W_A, keys_A, values_A, W_B, keys_B, values_B):
    xT = x.reshape(T, D).T
    WAf = W_A.reshape(C * CB_IN, D)
    WBf = W_B.reshape(C * CB_IN, D)
    vAT = values_A.transpose(0, 2, 1)
    vBg = (values_B.reshape(R, R, K, OUT_C)
           .transpose(1, 3, 0, 2).reshape(R, OUT_C, R * K))

    outT = pl.pallas_call(
        _tile_kernel,
        grid=(G,),
        in_specs=[
            pl.BlockSpec((D, TT), lambda i: (0, i)),
            pl.BlockSpec((C * CB_IN, D), lambda i: (0, 0)),
            pl.BlockSpec((C, K, CB_IN), lambda i: (0, 0, 0)),
            pl.BlockSpec((C, OUT_C, K), lambda i: (0, 0, 0)),
            pl.BlockSpec((C * CB_IN, D), lambda i: (0, 0)),
            pl.BlockSpec((C, K, CB_IN), lambda i: (0, 0, 0)),
            pl.BlockSpec((R, OUT_C, R * K), lambda i: (0, 0, 0)),
        ],
        out_specs=pl.BlockSpec((D, TT), lambda i: (0, i)),
        out_shape=jax.ShapeDtypeStruct((D, T), _F32),
    )(xT, WAf, keys_A, vAT, WBf, keys_B, vBg)

    return outT.T.reshape(B, N, D)
